# baseline (speedup 1.0000x reference)
import functools
import math

import jax
import jax.numpy as jnp
from jax import lax
from jax.experimental import pallas as pl
from jax.experimental.pallas import tpu as pltpu
from jax.experimental.pallas import tpu_sc as plsc

D_MODEL = 64
SCALE = math.sqrt(D_MODEL)

NUM_CORES = 2
NUM_SUBCORES = 16
NUM_WORKERS = NUM_CORES * NUM_SUBCORES
LANES = 16

CHUNK = 128


@functools.cache
def _emb_kernel(B: int):
    b_per_w = B // NUM_WORKERS
    n_groups = b_per_w // CHUNK
    mesh = plsc.VectorSubcoreMesh(core_axis_name="c", subcore_axis_name="s")

    @functools.partial(
        pl.kernel,
        out_type=jax.ShapeDtypeStruct((B, D_MODEL), jnp.float32),
        mesh=mesh,
        scratch_types=[
            pltpu.VMEM((CHUNK,), jnp.int32),
            pltpu.VMEM((CHUNK, D_MODEL), jnp.float32),
            pltpu.SemaphoreType.DMA,
        ],
        compiler_params=pltpu.CompilerParams(use_tc_tiling_on_sc=False),
    )
    def body(idx_hbm, table_hbm, out_hbm, idx_v, rows_v, sem):
        wid = lax.axis_index("s") * NUM_CORES + lax.axis_index("c")
        wbase = wid * b_per_w

        def group(g, carry):
            base = wbase + g * CHUNK
            pltpu.sync_copy(idx_hbm.at[pl.ds(base, CHUNK)], idx_v)
            pltpu.async_copy(table_hbm.at[idx_v], rows_v, sem).wait()

            def scale_row(i, c):
                for j in range(D_MODEL // LANES):
                    sl = pl.ds(j * LANES, LANES)
                    rows_v[i, sl] = rows_v[i, sl] * SCALE
                return c

            lax.fori_loop(0, CHUNK, scale_row, 0)
            pltpu.sync_copy(rows_v, out_hbm.at[pl.ds(base, CHUNK)])
            return carry

        lax.fori_loop(0, n_groups, group, 0)

    return body


def kernel(x, table):
    B = x.size
    flat = x.reshape(B).astype(jnp.int32)
    out = _emb_kernel(B)(flat, table)
    return out.reshape(x.shape + (D_MODEL,))


# speedup vs baseline: 1.2727x; 1.2727x over previous
import functools
import math

import jax
import jax.numpy as jnp
from jax import lax
from jax.experimental import pallas as pl
from jax.experimental.pallas import tpu as pltpu
from jax.experimental.pallas import tpu_sc as plsc

D_MODEL = 64
SCALE = math.sqrt(D_MODEL)

NUM_CORES = 2
NUM_SUBCORES = 16
NUM_WORKERS = NUM_CORES * NUM_SUBCORES
LANES = 16

CHUNK = 128
K = 2
NBUF = 4
DEPTH = 2
STEP_ROWS = K * CHUNK


@functools.cache
def _emb_kernel(B: int):
    b_per_w = B // NUM_WORKERS
    n_gathers = b_per_w // CHUNK
    n_steps = b_per_w // STEP_ROWS
    assert b_per_w % STEP_ROWS == 0 and n_steps % NBUF == 0
    mesh = plsc.VectorSubcoreMesh(core_axis_name="c", subcore_axis_name="s")

    @functools.partial(
        pl.kernel,
        out_type=jax.ShapeDtypeStruct((B, D_MODEL), jnp.float32),
        mesh=mesh,
        scratch_types=[
            pltpu.VMEM((n_gathers, CHUNK), jnp.int32),
        ]
        + [pltpu.VMEM((STEP_ROWS, D_MODEL), jnp.float32) for _ in range(NBUF)]
        + [pltpu.SemaphoreType.DMA for _ in range(2 * NBUF)],
        compiler_params=pltpu.CompilerParams(use_tc_tiling_on_sc=False),
    )
    def body(idx_hbm, table_hbm, out_hbm, idx_all, *bufs_and_sems):
        rows = bufs_and_sems[:NBUF]
        sem_g = bufs_and_sems[NBUF : 2 * NBUF]
        sem_s = bufs_and_sems[2 * NBUF : 3 * NBUF]

        wid = lax.axis_index("s") * NUM_CORES + lax.axis_index("c")
        wbase = wid * b_per_w

        def issue_gather(s, b):
            for k in range(K):
                j = s * K + k
                pltpu.async_copy(
                    table_hbm.at[idx_all.at[j]],
                    rows[b].at[pl.ds(k * CHUNK, CHUNK)],
                    sem_g[b],
                )

        def wait_gather(s, b):
            for k in range(K):
                j = s * K + k
                pltpu.make_async_copy(
                    table_hbm.at[idx_all.at[j]],
                    rows[b].at[pl.ds(k * CHUNK, CHUNK)],
                    sem_g[b],
                ).wait()

        def issue_scatter(s, b):
            obase = wbase + s * STEP_ROWS
            pltpu.async_copy(rows[b], out_hbm.at[pl.ds(obase, STEP_ROWS)], sem_s[b])

        def wait_scatter(b):
            pltpu.make_async_copy(
                rows[b], out_hbm.at[pl.ds(wbase, STEP_ROWS)], sem_s[b]
            ).wait()

        pltpu.sync_copy(idx_hbm.at[wid], idx_all)

        for s0 in range(DEPTH):
            issue_gather(s0, s0 % NBUF)

        @pl.loop(0, n_steps, step=NBUF)
        def outer(g):
            for b in range(NBUF):
                s = g + b
                wait_gather(s, b)

                @pl.loop(0, STEP_ROWS, unroll=4)
                def scale_row(i):
                    for j in range(D_MODEL // LANES):
                        sl = pl.ds(j * LANES, LANES)
                        rows[b][i, sl] = rows[b][i, sl] * SCALE

                issue_scatter(s, b)

                bn = (b + DEPTH) % NBUF

                @pl.when(s + DEPTH < n_steps)
                def _():
                    @pl.when(s + DEPTH >= NBUF)
                    def _():
                        wait_scatter(bn)

                    issue_gather(s + DEPTH, bn)

        for b in range(NBUF):
            wait_scatter(b)

    return body


def kernel(x, table):
    B = x.size
    b_per_w = B // NUM_WORKERS
    idx3 = x.reshape(NUM_WORKERS, b_per_w // CHUNK, CHUNK).astype(jnp.int32)
    out = _emb_kernel(B)(idx3, table)
    return out.reshape(x.shape + (D_MODEL,))
